# baseline (speedup 1.0000x reference)
import functools

import jax
import jax.numpy as jnp
from jax import lax
from jax.experimental import pallas as pl
from jax.experimental.pallas import tpu as pltpu
from jax.experimental.pallas import tpu_sc as plsc

NUM_BITS = 16
THETA1 = 4.0
THETA2 = 4.0

_NC = 2
_NS = 16
_NW = _NC * _NS


def _sc_densify(idx, sign, n_rows, n_cols, k_syn, chunk_rows):
    rows_per_w = n_rows // _NW
    n_chunks = rows_per_w // chunk_rows
    n_groups = k_syn // 16
    mesh = plsc.VectorSubcoreMesh(core_axis_name="c", subcore_axis_name="s")

    @functools.partial(
        pl.kernel,
        out_type=jax.ShapeDtypeStruct((n_rows, n_cols), jnp.float32),
        mesh=mesh,
        compiler_params=pltpu.CompilerParams(
            needs_layout_passes=False, use_tc_tiling_on_sc=True),
        scratch_types=[
            pltpu.VMEM((chunk_rows, n_cols), jnp.float32),
            pltpu.VMEM((chunk_rows, n_cols), jnp.float32),
            pltpu.VMEM((rows_per_w * k_syn,), jnp.int32),
            pltpu.VMEM((rows_per_w * k_syn,), jnp.float32),
            pltpu.SemaphoreType.DMA,
            pltpu.SemaphoreType.DMA,
        ],
    )
    def dens(idx_hbm, sign_hbm, w_hbm, buf_a, buf_b, idxv, sgnv, sem_a, sem_b):
        wid = lax.axis_index("s") * _NC + lax.axis_index("c")
        row_base = wid * rows_per_w
        syn_base = row_base * k_syn
        pltpu.sync_copy(idx_hbm.at[pl.ds(syn_base, rows_per_w * k_syn)], idxv)
        pltpu.sync_copy(sign_hbm.at[pl.ds(syn_base, rows_per_w * k_syn)], sgnv)
        zeros16 = jnp.zeros((16,), jnp.float32)
        bufs = (buf_a, buf_b)
        sems = (sem_a, sem_b)

        def zero_body(i, carry):
            r = i // (n_cols // 64)
            j = i % (n_cols // 64)
            for u in range(4):
                buf_a[r, pl.ds(j * 64 + u * 16, 16)] = zeros16
                buf_b[r, pl.ds(j * 64 + u * 16, 16)] = zeros16
            return carry

        lax.fori_loop(0, chunk_rows * n_cols // 64, zero_body, 0)

        def scatter_chunk(c, buf):
            row0 = c * chunk_rows
            for r in range(chunk_rows):
                rv = jnp.full((16,), r, jnp.int32)
                for g in range(n_groups):
                    s = (row0 + r) * k_syn + g * 16
                    iv = idxv[pl.ds(s, 16)]
                    sv = sgnv[pl.ds(s, 16)]
                    plsc.addupdate_scatter(buf, [rv, iv], sv)

        def restore_chunk(c, buf):
            row0 = c * chunk_rows
            for r in range(chunk_rows):
                rv = jnp.full((16,), r, jnp.int32)
                for g in range(n_groups):
                    s = (row0 + r) * k_syn + g * 16
                    iv = idxv[pl.ds(s, 16)]
                    plsc.store_scatter(buf, [rv, iv], zeros16)

        def start_dma(c, buf, sem):
            return pltpu.async_copy(
                buf, w_hbm.at[pl.ds(row_base + c * chunk_rows, chunk_rows)],
                sem)

        handles = [None, None]
        scatter_chunk(0, bufs[0])
        handles[0] = start_dma(0, bufs[0], sems[0])
        for c in range(1, n_chunks):
            p, q = c % 2, (c - 1) % 2
            scatter_chunk(c, bufs[p])
            handles[q].wait()
            restore_chunk(c - 1, bufs[q])
            handles[p] = start_dma(c, bufs[p], sems[p])
        handles[(n_chunks - 1) % 2].wait()

    return dens(idx, sign)


def _tc_encode(x_t_pad, thr_col):
    fp, b = x_t_pad.shape
    ep = fp * NUM_BITS

    def body(x_ref, t_ref, o_ref):
        xp = x_ref[...]
        xe = jnp.broadcast_to(xp[:, None, :], (fp, NUM_BITS, b)).reshape(ep, b)
        o_ref[...] = (xe > t_ref[...]).astype(jnp.bfloat16)

    return pl.pallas_call(
        body,
        out_shape=jax.ShapeDtypeStruct((ep, b), jnp.bfloat16),
    )(x_t_pad, thr_col)


def _tc_layer(w, act, y_col, theta, c_pad, block_h):
    h, d = w.shape
    b = act.shape[1]

    def body(w_ref, a_ref, y_ref, o_ref, uq_ref, ur_ref):
        wb = w_ref[...].astype(jnp.bfloat16)
        z = jnp.dot(wb, a_ref[...], preferred_element_type=jnp.float32)
        a_blk = (z >= theta).astype(jnp.bfloat16)
        o_ref[...] = a_blk
        cls = lax.broadcasted_iota(jnp.int32, (b, c_pad), 1)
        onehot = (y_ref[...] == cls).astype(jnp.bfloat16)
        u = jnp.dot(a_blk, onehot, preferred_element_type=jnp.float32)
        uq = jnp.floor(u * (1.0 / 256.0))
        uq_ref[...] = uq.astype(jnp.bfloat16)
        ur_ref[...] = (u - uq * 256.0).astype(jnp.bfloat16)

    return pl.pallas_call(
        body,
        grid=(h // block_h,),
        in_specs=[
            pl.BlockSpec((block_h, d), lambda i: (i, 0)),
            pl.BlockSpec((d, b), lambda i: (0, 0)),
            pl.BlockSpec((b, 1), lambda i: (0, 0)),
        ],
        out_specs=[
            pl.BlockSpec((block_h, b), lambda i: (i, 0)),
            pl.BlockSpec((block_h, c_pad), lambda i: (i, 0)),
            pl.BlockSpec((block_h, c_pad), lambda i: (i, 0)),
        ],
        out_shape=[
            jax.ShapeDtypeStruct((h, b), jnp.bfloat16),
            jax.ShapeDtypeStruct((h, c_pad), jnp.bfloat16),
            jax.ShapeDtypeStruct((h, c_pad), jnp.bfloat16),
        ],
    )(w, act, y_col)


def _tc_scores(lhs, a, c_pad, block_h):
    m, h = lhs.shape
    b = a.shape[1]

    def body(l_ref, a_ref, s_ref):
        @pl.when(pl.program_id(0) == 0)
        def _():
            s_ref[...] = jnp.zeros_like(s_ref)

        o = jnp.dot(l_ref[...], a_ref[...], preferred_element_type=jnp.float32)
        s_ref[...] += (o[0:c_pad] + o[c_pad:2 * c_pad]
                       + 256.0 * o[2 * c_pad:3 * c_pad] + o[3 * c_pad:])

    return pl.pallas_call(
        body,
        grid=(h // block_h,),
        in_specs=[
            pl.BlockSpec((m, block_h), lambda i: (0, i)),
            pl.BlockSpec((block_h, b), lambda i: (i, 0)),
        ],
        out_specs=pl.BlockSpec((c_pad, b), lambda i: (0, 0)),
        out_shape=jax.ShapeDtypeStruct((c_pad, b), jnp.float32),
    )(lhs, a)


def kernel(x, y, idx1, sign1, idx2, sign2, outConn):
    b, f = x.shape
    h, k_syn = idx1.shape
    c = outConn.shape[-1]
    e = f * NUM_BITS

    f_pad = ((f + 7) // 8) * 8
    e_pad = f_pad * NUM_BITS
    c_pad = 16

    x_t = jnp.pad(x.T, ((0, f_pad - f), (0, 0)))
    thr = jnp.linspace(0.0, 1.0, NUM_BITS, dtype=jnp.float32)
    thr_col = jnp.pad(jnp.tile(thr, f), (0, e_pad - e),
                      constant_values=2.0).reshape(e_pad, 1)
    y_col = y.reshape(b, 1)
    oc_t = jnp.pad(outConn.transpose(0, 2, 1), ((0, 0), (0, c_pad - c), (0, 0)))
    oc_h = oc_t.astype(jnp.bfloat16)
    oc_l = (oc_t - oc_h.astype(jnp.float32)).astype(jnp.bfloat16)

    w1t = _sc_densify(idx1.reshape(-1), sign1.reshape(-1), h, e_pad, k_syn, 16)
    w2t = _sc_densify(idx2.reshape(-1), sign2.reshape(-1), h, h, k_syn, 8)

    enc_t = _tc_encode(x_t, thr_col)
    a1, uq1, ur1 = _tc_layer(w1t, enc_t, y_col, THETA1, c_pad, 256)
    a2, uq2, ur2 = _tc_layer(w2t, a1, y_col, THETA2, c_pad, 256)
    lhs1 = jnp.concatenate([oc_h[0], oc_l[0], uq1.T, ur1.T], axis=0)
    lhs2 = jnp.concatenate([oc_h[1], oc_l[1], uq2.T, ur2.T], axis=0)
    s_t1 = _tc_scores(lhs1, a1, c_pad, 256)
    s_t2 = _tc_scores(lhs2, a2, c_pad, 256)
    return (s_t1 + s_t2)[:c, :].T


# speedup vs baseline: 1.1532x; 1.1532x over previous
import functools

import jax
import jax.numpy as jnp
from jax import lax
from jax.experimental import pallas as pl
from jax.experimental.pallas import tpu as pltpu
from jax.experimental.pallas import tpu_sc as plsc

NUM_BITS = 16
THETA1 = 4.0
THETA2 = 4.0

_NC = 2
_NS = 16
_NW = _NC * _NS


def _sc_densify(idx, sign, n_rows, n_cols, k_syn, chunk_rows):
    rows_per_w = n_rows // _NW
    n_chunks = rows_per_w // chunk_rows
    n_groups = k_syn // 16
    mesh = plsc.VectorSubcoreMesh(core_axis_name="c", subcore_axis_name="s")

    @functools.partial(
        pl.kernel,
        out_type=jax.ShapeDtypeStruct((n_rows, n_cols), jnp.float32),
        mesh=mesh,
        compiler_params=pltpu.CompilerParams(
            needs_layout_passes=False, use_tc_tiling_on_sc=True),
        scratch_types=[
            pltpu.VMEM((chunk_rows, n_cols), jnp.float32),
            pltpu.VMEM((chunk_rows, n_cols), jnp.float32),
            pltpu.VMEM((rows_per_w * k_syn,), jnp.int32),
            pltpu.VMEM((rows_per_w * k_syn,), jnp.float32),
            pltpu.SemaphoreType.DMA,
            pltpu.SemaphoreType.DMA,
        ],
    )
    def dens(idx_hbm, sign_hbm, w_hbm, buf_a, buf_b, idxv, sgnv, sem_a, sem_b):
        wid = lax.axis_index("s") * _NC + lax.axis_index("c")
        row_base = wid * rows_per_w
        syn_base = row_base * k_syn
        pltpu.sync_copy(idx_hbm.at[pl.ds(syn_base, rows_per_w * k_syn)], idxv)
        pltpu.sync_copy(sign_hbm.at[pl.ds(syn_base, rows_per_w * k_syn)], sgnv)
        zeros16 = jnp.zeros((16,), jnp.float32)
        bufs = (buf_a, buf_b)
        sems = (sem_a, sem_b)

        def zero_body(i, carry):
            r = i // (n_cols // 64)
            j = i % (n_cols // 64)
            for u in range(4):
                buf_a[r, pl.ds(j * 64 + u * 16, 16)] = zeros16
                buf_b[r, pl.ds(j * 64 + u * 16, 16)] = zeros16
            return carry

        lax.fori_loop(0, chunk_rows * n_cols // 64, zero_body, 0)

        def scatter_chunk(c, buf):
            row0 = c * chunk_rows
            for r in range(chunk_rows):
                rv = jnp.full((16,), r, jnp.int32)
                for g in range(n_groups):
                    s = (row0 + r) * k_syn + g * 16
                    iv = idxv[pl.ds(s, 16)]
                    sv = sgnv[pl.ds(s, 16)]
                    plsc.addupdate_scatter(buf, [rv, iv], sv)

        def restore_chunk(c, buf):
            row0 = c * chunk_rows
            for r in range(chunk_rows):
                rv = jnp.full((16,), r, jnp.int32)
                for g in range(n_groups):
                    s = (row0 + r) * k_syn + g * 16
                    iv = idxv[pl.ds(s, 16)]
                    plsc.store_scatter(buf, [rv, iv], zeros16)

        def start_dma(c, buf, sem):
            return pltpu.async_copy(
                buf, w_hbm.at[pl.ds(row_base + c * chunk_rows, chunk_rows)],
                sem)

        handles = [None, None]
        scatter_chunk(0, bufs[0])
        handles[0] = start_dma(0, bufs[0], sems[0])
        for c in range(1, n_chunks):
            p, q = c % 2, (c - 1) % 2
            scatter_chunk(c, bufs[p])
            handles[q].wait()
            restore_chunk(c - 1, bufs[q])
            handles[p] = start_dma(c, bufs[p], sems[p])
        handles[(n_chunks - 1) % 2].wait()

    return dens(idx, sign)


def _tc_encode(x_t_pad, thr_col):
    fp, b = x_t_pad.shape
    ep = fp * NUM_BITS

    def body(x_ref, t_ref, o_ref):
        xp = x_ref[...]
        xe = jnp.broadcast_to(xp[:, None, :], (fp, NUM_BITS, b)).reshape(ep, b)
        o_ref[...] = (xe > t_ref[...]).astype(jnp.bfloat16)

    return pl.pallas_call(
        body,
        out_shape=jax.ShapeDtypeStruct((ep, b), jnp.bfloat16),
    )(x_t_pad, thr_col)


def _tc_layer(w, act, y_col, theta, c_pad, block_h):
    h, d = w.shape
    b = act.shape[1]

    def body(w_ref, a_ref, y_ref, o_ref, uq_ref, ur_ref):
        wb = w_ref[...].astype(jnp.bfloat16)
        z = jnp.dot(wb, a_ref[...], preferred_element_type=jnp.float32)
        a_blk = (z >= theta).astype(jnp.bfloat16)
        o_ref[...] = a_blk
        cls = lax.broadcasted_iota(jnp.int32, (b, c_pad), 1)
        onehot = (y_ref[...] == cls).astype(jnp.bfloat16)
        u = jnp.dot(a_blk, onehot, preferred_element_type=jnp.float32)
        ut = u.T
        uq = jnp.floor(ut * (1.0 / 256.0))
        uq_ref[...] = uq.astype(jnp.bfloat16)
        ur_ref[...] = (ut - uq * 256.0).astype(jnp.bfloat16)

    return pl.pallas_call(
        body,
        grid=(h // block_h,),
        in_specs=[
            pl.BlockSpec((block_h, d), lambda i: (i, 0)),
            pl.BlockSpec((d, b), lambda i: (0, 0)),
            pl.BlockSpec((b, 1), lambda i: (0, 0)),
        ],
        out_specs=[
            pl.BlockSpec((block_h, b), lambda i: (i, 0)),
            pl.BlockSpec((c_pad, block_h), lambda i: (0, i)),
            pl.BlockSpec((c_pad, block_h), lambda i: (0, i)),
        ],
        out_shape=[
            jax.ShapeDtypeStruct((h, b), jnp.bfloat16),
            jax.ShapeDtypeStruct((c_pad, h), jnp.bfloat16),
            jax.ShapeDtypeStruct((c_pad, h), jnp.bfloat16),
        ],
    )(w, act, y_col)


def _tc_scores(och0, ocl0, uqt1, urt1, och1, ocl1, uqt2, urt2, a1, a2,
               c_pad, block_h):
    h = uqt1.shape[1]
    b = a1.shape[1]

    def one_layer(oh, ol, uq, ur, ab):
        lhs = jnp.concatenate([oh, ol, uq, ur], axis=0)
        o = jnp.dot(lhs, ab, preferred_element_type=jnp.float32)
        return (o[0:c_pad] + o[c_pad:2 * c_pad]
                + 256.0 * o[2 * c_pad:3 * c_pad] + o[3 * c_pad:])

    def body(oh0_ref, ol0_ref, uq1_ref, ur1_ref, oh1_ref, ol1_ref,
             uq2_ref, ur2_ref, a1_ref, a2_ref, s_ref):
        @pl.when(pl.program_id(0) == 0)
        def _():
            s_ref[...] = jnp.zeros_like(s_ref)

        s_ref[...] += (
            one_layer(oh0_ref[...], ol0_ref[...], uq1_ref[...], ur1_ref[...],
                      a1_ref[...])
            + one_layer(oh1_ref[...], ol1_ref[...], uq2_ref[...], ur2_ref[...],
                        a2_ref[...]))

    row_spec = pl.BlockSpec((c_pad, block_h), lambda i: (0, i))
    a_spec = pl.BlockSpec((block_h, b), lambda i: (i, 0))
    return pl.pallas_call(
        body,
        grid=(h // block_h,),
        in_specs=[row_spec, row_spec, row_spec, row_spec,
                  row_spec, row_spec, row_spec, row_spec, a_spec, a_spec],
        out_specs=pl.BlockSpec((c_pad, b), lambda i: (0, 0)),
        out_shape=jax.ShapeDtypeStruct((c_pad, b), jnp.float32),
    )(och0, ocl0, uqt1, urt1, och1, ocl1, uqt2, urt2, a1, a2)


def kernel(x, y, idx1, sign1, idx2, sign2, outConn):
    b, f = x.shape
    h, k_syn = idx1.shape
    c = outConn.shape[-1]
    e = f * NUM_BITS

    f_pad = ((f + 7) // 8) * 8
    e_pad = f_pad * NUM_BITS
    c_pad = 16

    x_t = jnp.pad(x.T, ((0, f_pad - f), (0, 0)))
    thr = jnp.linspace(0.0, 1.0, NUM_BITS, dtype=jnp.float32)
    thr_col = jnp.pad(jnp.tile(thr, f), (0, e_pad - e),
                      constant_values=2.0).reshape(e_pad, 1)
    y_col = y.reshape(b, 1)
    oc_t = jnp.pad(outConn.transpose(0, 2, 1), ((0, 0), (0, c_pad - c), (0, 0)))
    oc_h = oc_t.astype(jnp.bfloat16)
    oc_l = (oc_t - oc_h.astype(jnp.float32)).astype(jnp.bfloat16)

    w1t = _sc_densify(idx1.reshape(-1), sign1.reshape(-1), h, e_pad, k_syn, 16)
    w2t = _sc_densify(idx2.reshape(-1), sign2.reshape(-1), h, h, k_syn, 8)

    enc_t = _tc_encode(x_t, thr_col)
    a1, uq1, ur1 = _tc_layer(w1t, enc_t, y_col, THETA1, c_pad, 512)
    a2, uq2, ur2 = _tc_layer(w2t, a1, y_col, THETA2, c_pad, 512)
    s_t = _tc_scores(oc_h[0], oc_l[0], uq1, ur1, oc_h[1], oc_l[1], uq2, ur2,
                     a1, a2, c_pad, 1024)
    return s_t[:c, :].T


# speedup vs baseline: 1.2424x; 1.0774x over previous
import functools

import jax
import jax.numpy as jnp
from jax import lax
from jax.experimental import pallas as pl
from jax.experimental.pallas import tpu as pltpu
from jax.experimental.pallas import tpu_sc as plsc

NUM_BITS = 16
THETA1 = 4.0
THETA2 = 4.0

_NC = 2
_NS = 16
_NW = _NC * _NS


_BIAS = 512
_INIT_WORD = (_BIAS << 16) | _BIAS


def _sc_densify(idx, sign, n_rows, n_cols, k_syn, chunk_rows):
    n_half = n_cols // 2
    rows_per_w = n_rows // _NW
    n_chunks = rows_per_w // chunk_rows
    n_groups = k_syn // 16
    mesh = plsc.VectorSubcoreMesh(core_axis_name="c", subcore_axis_name="s")

    @functools.partial(
        pl.kernel,
        out_type=jax.ShapeDtypeStruct((n_rows, n_half), jnp.int32),
        mesh=mesh,
        compiler_params=pltpu.CompilerParams(
            needs_layout_passes=False, use_tc_tiling_on_sc=True),
        scratch_types=[
            pltpu.VMEM((chunk_rows, n_half), jnp.int32),
            pltpu.VMEM((chunk_rows, n_half), jnp.int32),
            pltpu.VMEM((rows_per_w * k_syn,), jnp.int32),
            pltpu.VMEM((rows_per_w * k_syn,), jnp.float32),
            pltpu.SemaphoreType.DMA,
            pltpu.SemaphoreType.DMA,
        ],
    )
    def dens(idx_hbm, sign_hbm, w_hbm, buf_a, buf_b, idxv, sgnv, sem_a, sem_b):
        wid = lax.axis_index("s") * _NC + lax.axis_index("c")
        row_base = wid * rows_per_w
        syn_base = row_base * k_syn
        pltpu.sync_copy(idx_hbm.at[pl.ds(syn_base, rows_per_w * k_syn)], idxv)
        pltpu.sync_copy(sign_hbm.at[pl.ds(syn_base, rows_per_w * k_syn)], sgnv)
        init16 = jnp.full((16,), _INIT_WORD, jnp.int32)
        bufs = (buf_a, buf_b)
        sems = (sem_a, sem_b)

        def zero_body(i, carry):
            r = i // (n_half // 64)
            j = i % (n_half // 64)
            for u in range(4):
                buf_a[r, pl.ds(j * 64 + u * 16, 16)] = init16
                buf_b[r, pl.ds(j * 64 + u * 16, 16)] = init16
            return carry

        lax.fori_loop(0, chunk_rows * n_half // 64, zero_body, 0)

        def split(s):
            iv = idxv[pl.ds(s, 16)]
            si = sgnv[pl.ds(s, 16)].astype(jnp.int32)
            hi = iv >= n_half
            wv = jnp.where(hi, iv - n_half, iv)
            sv = jnp.where(hi, si * 65536, si)
            return wv, sv

        def scatter_chunk(c, buf):
            row0 = c * chunk_rows
            for r in range(chunk_rows):
                rv = jnp.full((16,), r, jnp.int32)
                for g in range(n_groups):
                    wv, sv = split((row0 + r) * k_syn + g * 16)
                    plsc.addupdate_scatter(buf, [rv, wv], sv)

        def restore_chunk(c, buf):
            row0 = c * chunk_rows
            for r in range(chunk_rows):
                rv = jnp.full((16,), r, jnp.int32)
                for g in range(n_groups):
                    wv, _ = split((row0 + r) * k_syn + g * 16)
                    plsc.store_scatter(buf, [rv, wv], init16)

        def start_dma(c, buf, sem):
            return pltpu.async_copy(
                buf, w_hbm.at[pl.ds(row_base + c * chunk_rows, chunk_rows)],
                sem)

        handles = [None, None]
        scatter_chunk(0, bufs[0])
        handles[0] = start_dma(0, bufs[0], sems[0])
        for c in range(1, n_chunks):
            p, q = c % 2, (c - 1) % 2
            scatter_chunk(c, bufs[p])
            handles[q].wait()
            restore_chunk(c - 1, bufs[q])
            handles[p] = start_dma(c, bufs[p], sems[p])
        handles[(n_chunks - 1) % 2].wait()

    return dens(idx, sign)


def _tc_encode(x_t_pad, thr_col):
    fp, b = x_t_pad.shape
    ep = fp * NUM_BITS

    def body(x_ref, t_ref, o_ref):
        xp = x_ref[...]
        xe = jnp.broadcast_to(xp[:, None, :], (fp, NUM_BITS, b)).reshape(ep, b)
        o_ref[...] = (xe > t_ref[...]).astype(jnp.bfloat16)

    return pl.pallas_call(
        body,
        out_shape=jax.ShapeDtypeStruct((ep, b), jnp.bfloat16),
    )(x_t_pad, thr_col)


def _tc_layer(w, act, y_col, theta, c_pad, block_h):
    h, dh = w.shape
    d, b = act.shape

    def body(w_ref, a_ref, y_ref, o_ref, uq_ref, ur_ref):
        wi = w_ref[...]
        w_lo = ((wi & 0xFFFF) - _BIAS).astype(jnp.bfloat16)
        w_hi = ((wi >> 16) - _BIAS).astype(jnp.bfloat16)
        dh = wi.shape[1]
        z = (jnp.dot(w_lo, a_ref[0:dh, :], preferred_element_type=jnp.float32)
             + jnp.dot(w_hi, a_ref[dh:, :], preferred_element_type=jnp.float32))
        a_blk = (z >= theta).astype(jnp.bfloat16)
        o_ref[...] = a_blk
        cls = lax.broadcasted_iota(jnp.int32, (b, c_pad), 1)
        onehot = (y_ref[...] == cls).astype(jnp.bfloat16)
        u = jnp.dot(a_blk, onehot, preferred_element_type=jnp.float32)
        ut = u.T
        uq = jnp.floor(ut * (1.0 / 256.0))
        uq_ref[...] = uq.astype(jnp.bfloat16)
        ur_ref[...] = (ut - uq * 256.0).astype(jnp.bfloat16)

    return pl.pallas_call(
        body,
        grid=(h // block_h,),
        in_specs=[
            pl.BlockSpec((block_h, dh), lambda i: (i, 0)),
            pl.BlockSpec((d, b), lambda i: (0, 0)),
            pl.BlockSpec((b, 1), lambda i: (0, 0)),
        ],
        out_specs=[
            pl.BlockSpec((block_h, b), lambda i: (i, 0)),
            pl.BlockSpec((c_pad, block_h), lambda i: (0, i)),
            pl.BlockSpec((c_pad, block_h), lambda i: (0, i)),
        ],
        out_shape=[
            jax.ShapeDtypeStruct((h, b), jnp.bfloat16),
            jax.ShapeDtypeStruct((c_pad, h), jnp.bfloat16),
            jax.ShapeDtypeStruct((c_pad, h), jnp.bfloat16),
        ],
    )(w, act, y_col)


def _tc_scores(och0, ocl0, uqt1, urt1, och1, ocl1, uqt2, urt2, a1, a2,
               c_pad, block_h):
    h = uqt1.shape[1]
    b = a1.shape[1]

    def one_layer(oh, ol, uq, ur, ab):
        lhs = jnp.concatenate([oh, ol, uq, ur], axis=0)
        o = jnp.dot(lhs, ab, preferred_element_type=jnp.float32)
        return (o[0:c_pad] + o[c_pad:2 * c_pad]
                + 256.0 * o[2 * c_pad:3 * c_pad] + o[3 * c_pad:])

    def body(oh0_ref, ol0_ref, uq1_ref, ur1_ref, oh1_ref, ol1_ref,
             uq2_ref, ur2_ref, a1_ref, a2_ref, s_ref):
        @pl.when(pl.program_id(0) == 0)
        def _():
            s_ref[...] = jnp.zeros_like(s_ref)

        s_ref[...] += (
            one_layer(oh0_ref[...], ol0_ref[...], uq1_ref[...], ur1_ref[...],
                      a1_ref[...])
            + one_layer(oh1_ref[...], ol1_ref[...], uq2_ref[...], ur2_ref[...],
                        a2_ref[...]))

    row_spec = pl.BlockSpec((c_pad, block_h), lambda i: (0, i))
    a_spec = pl.BlockSpec((block_h, b), lambda i: (i, 0))
    return pl.pallas_call(
        body,
        grid=(h // block_h,),
        in_specs=[row_spec, row_spec, row_spec, row_spec,
                  row_spec, row_spec, row_spec, row_spec, a_spec, a_spec],
        out_specs=pl.BlockSpec((c_pad, b), lambda i: (0, 0)),
        out_shape=jax.ShapeDtypeStruct((c_pad, b), jnp.float32),
    )(och0, ocl0, uqt1, urt1, och1, ocl1, uqt2, urt2, a1, a2)


def kernel(x, y, idx1, sign1, idx2, sign2, outConn):
    b, f = x.shape
    h, k_syn = idx1.shape
    c = outConn.shape[-1]
    e = f * NUM_BITS

    f_pad = ((f + 7) // 8) * 8
    e_pad = f_pad * NUM_BITS
    c_pad = 16

    x_t = jnp.pad(x.T, ((0, f_pad - f), (0, 0)))
    thr = jnp.linspace(0.0, 1.0, NUM_BITS, dtype=jnp.float32)
    thr_col = jnp.pad(jnp.tile(thr, f), (0, e_pad - e),
                      constant_values=2.0).reshape(e_pad, 1)
    y_col = y.reshape(b, 1)
    oc_t = jnp.pad(outConn.transpose(0, 2, 1), ((0, 0), (0, c_pad - c), (0, 0)))
    oc_h = oc_t.astype(jnp.bfloat16)
    oc_l = (oc_t - oc_h.astype(jnp.float32)).astype(jnp.bfloat16)

    w1t = _sc_densify(idx1.reshape(-1), sign1.reshape(-1), h, e_pad, k_syn, 32)
    w2t = _sc_densify(idx2.reshape(-1), sign2.reshape(-1), h, h, k_syn, 16)

    enc_t = _tc_encode(x_t, thr_col)
    a1, uq1, ur1 = _tc_layer(w1t, enc_t, y_col, THETA1, c_pad, 512)
    a2, uq2, ur2 = _tc_layer(w2t, a1, y_col, THETA2, c_pad, 512)
    s_t = _tc_scores(oc_h[0], oc_l[0], uq1, ur1, oc_h[1], oc_l[1], uq2, ur2,
                     a1, a2, c_pad, 1024)
    return s_t[:c, :].T
